# Initial kernel scaffold; baseline (speedup 1.0000x reference)
import jax
import jax.numpy as jnp
from jax.experimental import pallas as pl


def kernel(xn, xe, edge_index, KNopen, KEopen, KNclose, KN, KE):
    raise NotImplementedError("write your pallas kernel here")


# speedup vs baseline: 3.5726x; 3.5726x over previous
import functools

import jax
import jax.numpy as jnp
from jax import lax
from jax.experimental import pallas as pl
from jax.experimental.pallas import tpu as pltpu
from jax.experimental.pallas import tpu_sc as plsc

H = 0.1
EPS = 1e-3

NC = 2
NS = 16
NW = NC * NS

CH = 80
NSTR = 5
SUP = CH * NSTR

EB = 2560


def _mesh():
    return plsc.VectorSubcoreMesh(
        core_axis_name="c", subcore_axis_name="s", num_cores=NC, num_subcores=NS
    )


def _prologue_body(xn_ref, knopen_ref, kn0_ref, xn0_ref, y1t_ref):
    xn0 = lax.dot_general(knopen_ref[...], xn_ref[...], (((1,), (0,)), ((), ())),
                          preferred_element_type=jnp.float32)
    xn0_ref[...] = xn0
    y1t_ref[...] = lax.dot_general(xn0, kn0_ref[...], (((0,), (1,)), ((), ())),
                                   preferred_element_type=jnp.float32)


def _stats_mi(stats, e_total):
    s = jnp.sum(stats, axis=0)
    m = s[0] / e_total
    inv = lax.rsqrt(s[1] - e_total * m * m + EPS)
    return m, inv


def _update0_body(ai_ref, xeraw_ref, keopen_ref, stats_ref, out_ref, *, e_total):
    m, inv = _stats_mi(stats_ref[...], e_total)
    xe0 = lax.dot_general(xeraw_ref[...], keopen_ref[...], (((0,), (1,)), ((), ())),
                          preferred_element_type=jnp.float32)
    a = (ai_ref[...] - m[None, :]) * inv[None, :]
    out_ref[...] = xe0 + H * jnp.maximum(a, 0.0)


def _update_body(ai_ref, xe_ref, stats_ref, out_ref, *, e_total):
    m, inv = _stats_mi(stats_ref[...], e_total)
    a = (ai_ref[...] - m[None, :]) * inv[None, :]
    out_ref[...] = xe_ref[...] + H * jnp.maximum(a, 0.0)


def _node_body(div_ref, xn_ref, ke_ref, wnext_ref, xn_new_ref, nxt_ref, *, last):
    dsum = div_ref[0] + div_ref[1]
    bi = lax.dot_general(ke_ref[...], dsum, (((1,), (1,)), ((), ())),
                         preferred_element_type=jnp.float32)
    bi = jnp.maximum(bi, 0.0)
    mu = jnp.mean(bi, axis=1, keepdims=True)
    xc = bi - mu
    bn = xc * lax.rsqrt(jnp.sum(xc * xc, axis=1, keepdims=True) + EPS)
    xn_new = xn_ref[...] + H * jnp.maximum(bn, 0.0)
    xn_new_ref[...] = xn_new
    if last:
        nxt_ref[...] = lax.dot_general(wnext_ref[...], xn_new, (((1,), (0,)), ((), ())),
                                       preferred_element_type=jnp.float32)
    else:
        nxt_ref[...] = lax.dot_general(xn_new, wnext_ref[...], (((0,), (1,)), ((), ())),
                                       preferred_element_type=jnp.float32)


def _close_xe_body(xe_ref, kc_ref, out_ref):
    out_ref[...] = lax.dot_general(kc_ref[...], xe_ref[...], (((1,), (1,)), ((), ())),
                                   preferred_element_type=jnp.float32)


def _sc_gather_body(src1, dst1, table, ai_out, stats_out,
                    sidx, didx, rows_a, rows_b, statbuf, sem, *, epw, nit):
    c = lax.axis_index("c")
    s = lax.axis_index("s")
    wid = s * NC + c
    zero = jnp.zeros((16,), jnp.float32)

    def super_body(i, carry):
        eoff = wid * epw + i * SUP
        pltpu.sync_copy(src1.at[pl.ds(eoff, SUP)], sidx)
        pltpu.sync_copy(dst1.at[pl.ds(eoff, SUP)], didx)
        cps = []
        for t in range(NSTR):
            cps.append(pltpu.async_copy(table.at[sidx.at[pl.ds(t * CH, CH)]],
                                        rows_a.at[pl.ds(t * CH, CH)], sem))
            cps.append(pltpu.async_copy(table.at[didx.at[pl.ds(t * CH, CH)]],
                                        rows_b.at[pl.ds(t * CH, CH)], sem))
        for cp in cps:
            cp.wait()

        def row_body(r, cr):
            out = list(cr)
            for q in range(4):
                a = rows_a[r, pl.ds(q * 16, 16)]
                b = rows_b[r, pl.ds(q * 16, 16)]
                d = a - b
                rows_a[r, pl.ds(q * 16, 16)] = d
                out[q] = out[q] + d
                out[4 + q] = out[4 + q] + d * d
            return tuple(out)

        carry = lax.fori_loop(0, SUP, row_body, carry)
        pltpu.sync_copy(rows_a, ai_out.at[pl.ds(eoff, SUP)])
        return carry

    carry = lax.fori_loop(0, nit, super_body, (zero,) * 8)
    for q in range(4):
        statbuf[0, pl.ds(q * 16, 16)] = carry[q]
        statbuf[1, pl.ds(q * 16, 16)] = carry[4 + q]
    pltpu.sync_copy(statbuf, stats_out.at[wid])


def _sc_scatter_body(xe_t, src1, dst1, div_out,
                     sidx, didx, rows_a, rows_b, bounce, sem, shared_div,
                     *, epw, nit, n, zch):
    c = lax.axis_index("c")
    s = lax.axis_index("s")
    wid = s * NC + c
    zero = jnp.zeros((16,), jnp.float32)
    nzch = n // zch
    njz = (nzch + NS - 1) // NS

    def zrow(r, _):
        for q in range(4):
            bounce[r, pl.ds(q * 16, 16)] = zero
        return 0

    lax.fori_loop(0, zch, zrow, 0)

    def zchunk(j, _):
        ck = s + j * NS

        @pl.when(ck < nzch)
        def _():
            pltpu.sync_copy(bounce, shared_div.at[pl.ds(ck * zch, zch)])

        return 0

    lax.fori_loop(0, njz, zchunk, 0)
    plsc.subcore_barrier()

    def super_body(i, _):
        eoff = wid * epw + i * SUP
        for t in range(NSTR):
            pltpu.sync_copy(src1.at[pl.ds(eoff + t * CH, CH)], sidx.at[t])
            pltpu.sync_copy(dst1.at[pl.ds(eoff + t * CH, CH)], didx.at[t])
        pltpu.sync_copy(xe_t.at[pl.ds(eoff, SUP)], rows_a)

        def row_body(r, _):
            for q in range(4):
                rows_b[r, pl.ds(q * 16, 16)] = -rows_a[r, pl.ds(q * 16, 16)]
            return 0

        lax.fori_loop(0, SUP, row_body, 0)
        for t in range(NSTR):
            pltpu.sync_copy(rows_a.at[pl.ds(t * CH, CH)],
                            shared_div.at[sidx.at[t]], add=True)
            pltpu.sync_copy(rows_b.at[pl.ds(t * CH, CH)],
                            shared_div.at[didx.at[t]], add=True)
        return 0

    lax.fori_loop(0, nit, super_body, 0)
    plsc.subcore_barrier()

    def dchunk(j, _):
        ck = s + j * NS

        @pl.when(ck < nzch)
        def _():
            pltpu.sync_copy(shared_div.at[pl.ds(ck * zch, zch)], bounce)
            pltpu.sync_copy(bounce, div_out.at[c].at[pl.ds(ck * zch, zch)])

        return 0

    lax.fori_loop(0, njz, dchunk, 0)


def kernel(xn, xe, edge_index, KNopen, KEopen, KNclose, KN, KE):
    nin, n = xn.shape
    e = xe.shape[1]
    cdim = KNopen.shape[0]
    nlayer = KN.shape[0]
    epw = e // NW
    nit = epw // SUP
    zch = 80
    ge = e // EB
    f32 = jnp.float32

    src1 = edge_index[0]
    dst1 = edge_index[1]

    xn_cur, table = pl.pallas_call(
        _prologue_body,
        out_shape=(jax.ShapeDtypeStruct((cdim, n), f32),
                   jax.ShapeDtypeStruct((n, cdim), f32)),
    )(xn, KNopen, KN[0])

    mesh = _mesh()
    gather_call = functools.partial(
        pl.kernel,
        functools.partial(_sc_gather_body, epw=epw, nit=nit),
        out_type=(jax.ShapeDtypeStruct((e, cdim), f32),
                  jax.ShapeDtypeStruct((NW, 2, cdim), f32)),
        mesh=mesh,
        scratch_types=[
            pltpu.VMEM((SUP,), jnp.int32),
            pltpu.VMEM((SUP,), jnp.int32),
            pltpu.VMEM((SUP, cdim), f32),
            pltpu.VMEM((SUP, cdim), f32),
            pltpu.VMEM((2, cdim), f32),
            pltpu.SemaphoreType.DMA,
        ],
        compiler_params=pltpu.CompilerParams(use_tc_tiling_on_sc=False),
    )
    scatter_call = functools.partial(
        pl.kernel,
        functools.partial(_sc_scatter_body, epw=epw, nit=nit, n=n, zch=zch),
        out_type=jax.ShapeDtypeStruct((NC, n, cdim), f32),
        mesh=mesh,
        scratch_types=[
            pltpu.VMEM((NSTR, CH), jnp.int32),
            pltpu.VMEM((NSTR, CH), jnp.int32),
            pltpu.VMEM((SUP, cdim), f32),
            pltpu.VMEM((SUP, cdim), f32),
            pltpu.VMEM((zch, cdim), f32),
            pltpu.SemaphoreType.DMA,
            pltpu.VMEM_SHARED((n, cdim), f32),
        ],
        compiler_params=pltpu.CompilerParams(use_tc_tiling_on_sc=False),
    )

    xe_t = None
    for i in range(nlayer):
        last = i == nlayer - 1

        ai_t, stats = gather_call()(src1, dst1, table)

        stats_spec = pl.BlockSpec((NW, 2, cdim), lambda i_: (0, 0, 0))
        eb_spec = pl.BlockSpec((EB, cdim), lambda i_: (i_, 0))
        if i == 0:
            xe_t = pl.pallas_call(
                functools.partial(_update0_body, e_total=float(e)),
                grid=(ge,),
                in_specs=[
                    eb_spec,
                    pl.BlockSpec((nin, EB), lambda i_: (0, i_)),
                    pl.BlockSpec((cdim, nin), lambda i_: (0, 0)),
                    stats_spec,
                ],
                out_specs=eb_spec,
                out_shape=jax.ShapeDtypeStruct((e, cdim), f32),
            )(ai_t, xe, KEopen, stats)
        else:
            xe_t = pl.pallas_call(
                functools.partial(_update_body, e_total=float(e)),
                grid=(ge,),
                in_specs=[eb_spec, eb_spec, stats_spec],
                out_specs=eb_spec,
                out_shape=jax.ShapeDtypeStruct((e, cdim), f32),
            )(ai_t, xe_t, stats)

        div_parts = scatter_call()(xe_t, src1, dst1)

        wnext = KNclose if last else KN[i + 1]
        nxt_shape = (cdim, n) if last else (n, cdim)
        xn_cur, nxt = pl.pallas_call(
            functools.partial(_node_body, last=last),
            out_shape=(jax.ShapeDtypeStruct((cdim, n), f32),
                       jax.ShapeDtypeStruct(nxt_shape, f32)),
        )(div_parts, xn_cur, KE[i], wnext)
        if last:
            xn_out = nxt
        else:
            table = nxt

    xe_out = pl.pallas_call(
        _close_xe_body,
        grid=(ge,),
        in_specs=[pl.BlockSpec((EB, cdim), lambda i_: (i_, 0)),
                  pl.BlockSpec((cdim, cdim), lambda i_: (0, 0))],
        out_specs=pl.BlockSpec((cdim, EB), lambda i_: (0, i_)),
        out_shape=jax.ShapeDtypeStruct((cdim, e), f32),
    )(xe_t, KNclose)

    return (xn_out, xe_out)


# speedup vs baseline: 4.5215x; 1.2656x over previous
import functools

import jax
import jax.numpy as jnp
from jax import lax
from jax.experimental import pallas as pl
from jax.experimental.pallas import tpu as pltpu
from jax.experimental.pallas import tpu_sc as plsc

H = 0.1
EPS = 1e-3

NC = 2
NS = 16
NW = NC * NS

CH = 80
NSTR = 5
SUP = CH * NSTR

EBP = 1280


def _mesh():
    return plsc.VectorSubcoreMesh(
        core_axis_name="c", subcore_axis_name="s", num_cores=NC, num_subcores=NS
    )


def _prologue_body(xn_ref, knopen_ref, kn0_ref, xn0_ref, y1t_ref):
    xn0 = lax.dot_general(knopen_ref[...], xn_ref[...], (((1,), (0,)), ((), ())),
                          preferred_element_type=jnp.float32)
    xn0_ref[...] = xn0
    y1t_ref[...] = lax.dot_general(xn0, kn0_ref[...], (((0,), (1,)), ((), ())),
                                   preferred_element_type=jnp.float32)


def _stats_mi2(stats, e_total):
    s = jnp.sum(stats, axis=0)
    m = s[0] / e_total
    inv = lax.rsqrt(s[1] - e_total * m * m + EPS)
    return jnp.concatenate([m, m]), jnp.concatenate([inv, inv])


def _update0_body(ai_ref, xea_ref, xeb_ref, w2a_ref, w2b_ref, stats_ref, out_ref,
                  *, e_total):
    m2, inv2 = _stats_mi2(stats_ref[...], e_total)
    xe0 = lax.dot_general(xea_ref[...], w2a_ref[...], (((0,), (1,)), ((), ())),
                          preferred_element_type=jnp.float32)
    xe0 = xe0 + lax.dot_general(xeb_ref[...], w2b_ref[...], (((0,), (1,)), ((), ())),
                                preferred_element_type=jnp.float32)
    a = (ai_ref[...] - m2[None, :]) * inv2[None, :]
    out_ref[...] = xe0 + H * jnp.maximum(a, 0.0)


def _update_body(ai_ref, xe_ref, stats_ref, out_ref, *, e_total):
    m2, inv2 = _stats_mi2(stats_ref[...], e_total)
    a = (ai_ref[...] - m2[None, :]) * inv2[None, :]
    out_ref[...] = xe_ref[...] + H * jnp.maximum(a, 0.0)


def _node_body(div_ref, xn_ref, ke_ref, wnext_ref, xn_new_ref, nxt_ref, *, last):
    dsum = div_ref[0] + div_ref[1]
    bi = lax.dot_general(ke_ref[...], dsum, (((1,), (1,)), ((), ())),
                         preferred_element_type=jnp.float32)
    bi = jnp.maximum(bi, 0.0)
    mu = jnp.mean(bi, axis=1, keepdims=True)
    xc = bi - mu
    bn = xc * lax.rsqrt(jnp.sum(xc * xc, axis=1, keepdims=True) + EPS)
    xn_new = xn_ref[...] + H * jnp.maximum(bn, 0.0)
    xn_new_ref[...] = xn_new
    if last:
        nxt_ref[...] = lax.dot_general(wnext_ref[...], xn_new, (((1,), (0,)), ((), ())),
                                       preferred_element_type=jnp.float32)
    else:
        nxt_ref[...] = lax.dot_general(xn_new, wnext_ref[...], (((0,), (1,)), ((), ())),
                                       preferred_element_type=jnp.float32)


def _close_xe_body(xe_ref, kca_ref, kcb_ref, out_ref, *, gehalf):
    i = pl.program_id(0)
    w = jnp.where(i < gehalf, kca_ref[...], kcb_ref[...])
    out_ref[...] = lax.dot_general(w, xe_ref[...], (((1,), (1,)), ((), ())),
                                   preferred_element_type=jnp.float32)


def _sc_gather_body(src1, dst1, table, ai_out, stats_out,
                    sidx, didx, rows_a, rows_b, statbuf, sem, *, epw, nit):
    c = lax.axis_index("c")
    s = lax.axis_index("s")
    wid = s * NC + c
    zero = jnp.zeros((16,), jnp.float32)

    def super_body(i, carry):
        eoff = wid * epw + i * SUP
        pltpu.sync_copy(src1.at[pl.ds(eoff, SUP)], sidx)
        pltpu.sync_copy(dst1.at[pl.ds(eoff, SUP)], didx)
        cps = []
        for t in range(NSTR):
            cps.append(pltpu.async_copy(table.at[sidx.at[pl.ds(t * CH, CH)]],
                                        rows_a.at[pl.ds(t * CH, CH)], sem))
            cps.append(pltpu.async_copy(table.at[didx.at[pl.ds(t * CH, CH)]],
                                        rows_b.at[pl.ds(t * CH, CH)], sem))
        for cp in cps:
            cp.wait()

        def row_body(r, cr):
            out = list(cr)
            for q in range(4):
                a = rows_a[r, pl.ds(q * 16, 16)]
                b = rows_b[r, pl.ds(q * 16, 16)]
                d = a - b
                rows_a[r, pl.ds(q * 16, 16)] = d
                out[q] = out[q] + d
                out[4 + q] = out[4 + q] + d * d
            return tuple(out)

        carry = lax.fori_loop(0, SUP, row_body, carry)
        pltpu.sync_copy(rows_a, ai_out.at[pl.ds(eoff, SUP)])
        return carry

    carry = lax.fori_loop(0, nit, super_body, (zero,) * 8)
    for q in range(4):
        statbuf[0, pl.ds(q * 16, 16)] = carry[q]
        statbuf[1, pl.ds(q * 16, 16)] = carry[4 + q]
    pltpu.sync_copy(statbuf, stats_out.at[wid])


def _sc_scatter_body(xe_t, src1, dst1, div_out,
                     sidx, didx, rows_a, rows_b, bounce, sem, shared_div,
                     *, epw, nit, n, zch):
    c = lax.axis_index("c")
    s = lax.axis_index("s")
    wid = s * NC + c
    zero = jnp.zeros((16,), jnp.float32)
    nzch = n // zch
    njz = (nzch + NS - 1) // NS

    def zrow(r, _):
        for q in range(4):
            bounce[r, pl.ds(q * 16, 16)] = zero
        return 0

    lax.fori_loop(0, zch, zrow, 0)

    def zchunk(j, _):
        ck = s + j * NS

        @pl.when(ck < nzch)
        def _():
            pltpu.sync_copy(bounce, shared_div.at[pl.ds(ck * zch, zch)])

        return 0

    lax.fori_loop(0, njz, zchunk, 0)
    plsc.subcore_barrier()

    def super_body(i, _):
        eoff = wid * epw + i * SUP
        for t in range(NSTR):
            pltpu.sync_copy(src1.at[pl.ds(eoff + t * CH, CH)], sidx.at[t])
            pltpu.sync_copy(dst1.at[pl.ds(eoff + t * CH, CH)], didx.at[t])
        pltpu.sync_copy(xe_t.at[pl.ds(eoff, SUP)], rows_a)

        def row_body(r, _):
            for q in range(4):
                rows_b[r, pl.ds(q * 16, 16)] = -rows_a[r, pl.ds(q * 16, 16)]
            return 0

        lax.fori_loop(0, SUP, row_body, 0)
        for t in range(NSTR):
            pltpu.sync_copy(rows_a.at[pl.ds(t * CH, CH)],
                            shared_div.at[sidx.at[t]], add=True)
            pltpu.sync_copy(rows_b.at[pl.ds(t * CH, CH)],
                            shared_div.at[didx.at[t]], add=True)
        return 0

    lax.fori_loop(0, nit, super_body, 0)
    plsc.subcore_barrier()

    def dchunk(j, _):
        ck = s + j * NS

        @pl.when(ck < nzch)
        def _():
            pltpu.sync_copy(shared_div.at[pl.ds(ck * zch, zch)], bounce)
            pltpu.sync_copy(bounce, div_out.at[c].at[pl.ds(ck * zch, zch)])

        return 0

    lax.fori_loop(0, njz, dchunk, 0)


def kernel(xn, xe, edge_index, KNopen, KEopen, KNclose, KN, KE):
    nin, n = xn.shape
    e = xe.shape[1]
    cdim = KNopen.shape[0]
    nlayer = KN.shape[0]
    epw = e // NW
    nit = epw // SUP
    zch = 80
    eh = e // 2
    ge = eh // EBP
    f32 = jnp.float32

    src1 = jnp.transpose(edge_index[0].reshape(2, eh)).reshape(e)
    dst1 = jnp.transpose(edge_index[1].reshape(2, eh)).reshape(e)

    zc = jnp.zeros_like(KEopen)
    w2a = jnp.concatenate([KEopen, zc], axis=0)
    w2b = jnp.concatenate([zc, KEopen], axis=0)
    zk = jnp.zeros_like(KNclose)
    kca = jnp.concatenate([KNclose, zk], axis=1)
    kcb = jnp.concatenate([zk, KNclose], axis=1)

    xn_cur, table = pl.pallas_call(
        _prologue_body,
        out_shape=(jax.ShapeDtypeStruct((cdim, n), f32),
                   jax.ShapeDtypeStruct((n, cdim), f32)),
    )(xn, KNopen, KN[0])

    mesh = _mesh()
    gather_call = functools.partial(
        pl.kernel,
        functools.partial(_sc_gather_body, epw=epw, nit=nit),
        out_type=(jax.ShapeDtypeStruct((e, cdim), f32),
                  jax.ShapeDtypeStruct((NW, 2, cdim), f32)),
        mesh=mesh,
        scratch_types=[
            pltpu.VMEM((SUP,), jnp.int32),
            pltpu.VMEM((SUP,), jnp.int32),
            pltpu.VMEM((SUP, cdim), f32),
            pltpu.VMEM((SUP, cdim), f32),
            pltpu.VMEM((2, cdim), f32),
            pltpu.SemaphoreType.DMA,
        ],
        compiler_params=pltpu.CompilerParams(use_tc_tiling_on_sc=False),
    )
    scatter_call = functools.partial(
        pl.kernel,
        functools.partial(_sc_scatter_body, epw=epw, nit=nit, n=n, zch=zch),
        out_type=jax.ShapeDtypeStruct((NC, n, cdim), f32),
        mesh=mesh,
        scratch_types=[
            pltpu.VMEM((NSTR, CH), jnp.int32),
            pltpu.VMEM((NSTR, CH), jnp.int32),
            pltpu.VMEM((SUP, cdim), f32),
            pltpu.VMEM((SUP, cdim), f32),
            pltpu.VMEM((zch, cdim), f32),
            pltpu.SemaphoreType.DMA,
            pltpu.VMEM_SHARED((n, cdim), f32),
        ],
        compiler_params=pltpu.CompilerParams(use_tc_tiling_on_sc=False),
    )

    xe_p = None
    for i in range(nlayer):
        last = i == nlayer - 1

        ai_t, stats = gather_call()(src1, dst1, table)
        ai_p = ai_t.reshape(eh, 2 * cdim)

        stats_spec = pl.BlockSpec((NW, 2, cdim), lambda i_: (0, 0, 0))
        ebp_spec = pl.BlockSpec((EBP, 2 * cdim), lambda i_: (i_, 0))
        if i == 0:
            xe_p = pl.pallas_call(
                functools.partial(_update0_body, e_total=float(e)),
                grid=(ge,),
                in_specs=[
                    ebp_spec,
                    pl.BlockSpec((nin, EBP), lambda i_: (0, i_)),
                    pl.BlockSpec((nin, EBP), lambda i_: (0, i_ + ge)),
                    pl.BlockSpec((2 * cdim, nin), lambda i_: (0, 0)),
                    pl.BlockSpec((2 * cdim, nin), lambda i_: (0, 0)),
                    stats_spec,
                ],
                out_specs=ebp_spec,
                out_shape=jax.ShapeDtypeStruct((eh, 2 * cdim), f32),
            )(ai_p, xe, xe, w2a, w2b, stats)
        else:
            xe_p = pl.pallas_call(
                functools.partial(_update_body, e_total=float(e)),
                grid=(ge,),
                in_specs=[ebp_spec, ebp_spec, stats_spec],
                out_specs=ebp_spec,
                out_shape=jax.ShapeDtypeStruct((eh, 2 * cdim), f32),
            )(ai_p, xe_p, stats)

        div_parts = scatter_call()(xe_p.reshape(e, cdim), src1, dst1)

        wnext = KNclose if last else KN[i + 1]
        nxt_shape = (cdim, n) if last else (n, cdim)
        xn_cur, nxt = pl.pallas_call(
            functools.partial(_node_body, last=last),
            out_shape=(jax.ShapeDtypeStruct((cdim, n), f32),
                       jax.ShapeDtypeStruct(nxt_shape, f32)),
        )(div_parts, xn_cur, KE[i], wnext)
        if last:
            xn_out = nxt
        else:
            table = nxt

    xe_out = pl.pallas_call(
        functools.partial(_close_xe_body, gehalf=ge),
        grid=(2 * ge,),
        in_specs=[pl.BlockSpec((EBP, 2 * cdim), lambda i_: (i_ % ge, 0)),
                  pl.BlockSpec((cdim, 2 * cdim), lambda i_: (0, 0)),
                  pl.BlockSpec((cdim, 2 * cdim), lambda i_: (0, 0))],
        out_specs=pl.BlockSpec((cdim, EBP), lambda i_: (0, i_)),
        out_shape=jax.ShapeDtypeStruct((cdim, e), f32),
    )(xe_p, kca, kcb)

    return (xn_out, xe_out)


# speedup vs baseline: 5.1917x; 1.1482x over previous
import functools

import jax
import jax.numpy as jnp
from jax import lax
from jax.experimental import pallas as pl
from jax.experimental.pallas import tpu as pltpu
from jax.experimental.pallas import tpu_sc as plsc

H = 0.1
EPS = 1e-3

NC = 2
NS = 16
NW = NC * NS

PSUP = 200
CHH = 40
NSTRH = PSUP // CHH

EBP = 1280


def _mesh():
    return plsc.VectorSubcoreMesh(
        core_axis_name="c", subcore_axis_name="s", num_cores=NC, num_subcores=NS
    )


def _prologue_body(xn_ref, knopen_ref, kn0_ref, xn0_ref, y1t_ref):
    xn0 = lax.dot_general(knopen_ref[...], xn_ref[...], (((1,), (0,)), ((), ())),
                          preferred_element_type=jnp.float32)
    xn0_ref[...] = xn0
    y1t_ref[...] = lax.dot_general(xn0, kn0_ref[...], (((0,), (1,)), ((), ())),
                                   preferred_element_type=jnp.float32)


def _stats_mi2(stats, e_total):
    s = jnp.sum(stats, axis=0)
    m = s[0] / e_total
    inv = lax.rsqrt(s[1] - e_total * m * m + EPS)
    return jnp.concatenate([m, m]), jnp.concatenate([inv, inv])


def _update0_body(ai_ref, xea_ref, xeb_ref, w2a_ref, w2b_ref, stats_ref, out_ref,
                  *, e_total):
    m2, inv2 = _stats_mi2(stats_ref[...], e_total)
    xe0 = lax.dot_general(xea_ref[...], w2a_ref[...], (((0,), (1,)), ((), ())),
                          preferred_element_type=jnp.float32)
    xe0 = xe0 + lax.dot_general(xeb_ref[...], w2b_ref[...], (((0,), (1,)), ((), ())),
                                preferred_element_type=jnp.float32)
    a = (ai_ref[...] - m2[None, :]) * inv2[None, :]
    out_ref[...] = xe0 + H * jnp.maximum(a, 0.0)


def _update_close_body(ai_ref, xe_ref, stats_ref, kca_ref, kcb_ref,
                       out_ref, outa_ref, outb_ref, *, e_total):
    m2, inv2 = _stats_mi2(stats_ref[...], e_total)
    a = (ai_ref[...] - m2[None, :]) * inv2[None, :]
    xe2 = xe_ref[...] + H * jnp.maximum(a, 0.0)
    out_ref[...] = xe2
    outa_ref[...] = lax.dot_general(kca_ref[...], xe2, (((1,), (1,)), ((), ())),
                                    preferred_element_type=jnp.float32)
    outb_ref[...] = lax.dot_general(kcb_ref[...], xe2, (((1,), (1,)), ((), ())),
                                    preferred_element_type=jnp.float32)


def _node_body(div_ref, xn_ref, ke_ref, wnext_ref, xn_new_ref, nxt_ref, *, last):
    dsum = div_ref[0] + div_ref[1]
    bi = lax.dot_general(ke_ref[...], dsum, (((1,), (1,)), ((), ())),
                         preferred_element_type=jnp.float32)
    bi = jnp.maximum(bi, 0.0)
    mu = jnp.mean(bi, axis=1, keepdims=True)
    xc = bi - mu
    bn = xc * lax.rsqrt(jnp.sum(xc * xc, axis=1, keepdims=True) + EPS)
    xn_new = xn_ref[...] + H * jnp.maximum(bn, 0.0)
    xn_new_ref[...] = xn_new
    if last:
        nxt_ref[...] = lax.dot_general(wnext_ref[...], xn_new, (((1,), (0,)), ((), ())),
                                       preferred_element_type=jnp.float32)
    else:
        nxt_ref[...] = lax.dot_general(xn_new, wnext_ref[...], (((0,), (1,)), ((), ())),
                                       preferred_element_type=jnp.float32)


def _sc_gather_body(src1, dst1, table, ai_out, stats_out,
                    sidxa, didxa, sidxb, didxb,
                    srows_a, drows_a, srows_b, drows_b, statbuf, sem,
                    *, eh, hpw, nitp):
    c = lax.axis_index("c")
    s = lax.axis_index("s")
    wid = s * NC + c
    zero = jnp.zeros((16,), jnp.float32)

    def super_body(i, carry):
        pr = wid * hpw + i * PSUP
        pltpu.sync_copy(src1.at[pl.ds(pr, PSUP)], sidxa)
        pltpu.sync_copy(dst1.at[pl.ds(pr, PSUP)], didxa)
        pltpu.sync_copy(src1.at[pl.ds(eh + pr, PSUP)], sidxb)
        pltpu.sync_copy(dst1.at[pl.ds(eh + pr, PSUP)], didxb)
        cps = []
        for t in range(NSTRH):
            sl = pl.ds(t * CHH, CHH)
            cps.append(pltpu.async_copy(table.at[sidxa.at[sl]], srows_a.at[sl], sem))
            cps.append(pltpu.async_copy(table.at[didxa.at[sl]], drows_a.at[sl], sem))
            cps.append(pltpu.async_copy(table.at[sidxb.at[sl]], srows_b.at[sl], sem))
            cps.append(pltpu.async_copy(table.at[didxb.at[sl]], drows_b.at[sl], sem))
        for cp in cps:
            cp.wait()

        def row_body(r, cr):
            out = list(cr)
            for sb, db in ((srows_a, drows_a), (srows_b, drows_b)):
                for q in range(4):
                    d = sb[r, pl.ds(q * 16, 16)] - db[r, pl.ds(q * 16, 16)]
                    sb[r, pl.ds(q * 16, 16)] = d
                    out[q] = out[q] + d
                    out[4 + q] = out[4 + q] + d * d
            return tuple(out)

        carry = lax.fori_loop(0, PSUP, row_body, carry)
        pltpu.sync_copy(srows_a, ai_out.at[pl.ds(pr, PSUP), pl.ds(0, 64)])
        pltpu.sync_copy(srows_b, ai_out.at[pl.ds(pr, PSUP), pl.ds(64, 64)])
        return carry

    carry = lax.fori_loop(0, nitp, super_body, (zero,) * 8)
    for q in range(4):
        statbuf[0, pl.ds(q * 16, 16)] = carry[q]
        statbuf[1, pl.ds(q * 16, 16)] = carry[4 + q]
    pltpu.sync_copy(statbuf, stats_out.at[wid])


def _sc_scatter_body(xe_p, src2, dst2, div_out,
                     sidxa, didxa, sidxb, didxb,
                     rows_a, rows_b, nrows_a, nrows_b, bounce, sem, shared_div,
                     *, eh, hpw, nitp, n, zch):
    c = lax.axis_index("c")
    s = lax.axis_index("s")
    wid = s * NC + c
    zero = jnp.zeros((16,), jnp.float32)
    nzch = n // zch
    njz = (nzch + NS - 1) // NS

    def zrow(r, _):
        for q in range(4):
            bounce[r, pl.ds(q * 16, 16)] = zero
        return 0

    lax.fori_loop(0, zch, zrow, 0)

    def zchunk(j, _):
        ck = s + j * NS

        @pl.when(ck < nzch)
        def _():
            pltpu.sync_copy(bounce, shared_div.at[pl.ds(ck * zch, zch)])

        return 0

    lax.fori_loop(0, njz, zchunk, 0)
    plsc.subcore_barrier()

    def super_body(i, _):
        pr = wid * hpw + i * PSUP
        rra = pr // CHH
        rrb = (eh + pr) // CHH
        pltpu.sync_copy(src2.at[pl.ds(rra, NSTRH)], sidxa)
        pltpu.sync_copy(dst2.at[pl.ds(rra, NSTRH)], didxa)
        pltpu.sync_copy(src2.at[pl.ds(rrb, NSTRH)], sidxb)
        pltpu.sync_copy(dst2.at[pl.ds(rrb, NSTRH)], didxb)
        pltpu.sync_copy(xe_p.at[pl.ds(pr, PSUP), pl.ds(0, 64)], rows_a)
        pltpu.sync_copy(xe_p.at[pl.ds(pr, PSUP), pl.ds(64, 64)], rows_b)

        def row_body(r, _):
            for rb, nb in ((rows_a, nrows_a), (rows_b, nrows_b)):
                for q in range(4):
                    nb[r, pl.ds(q * 16, 16)] = -rb[r, pl.ds(q * 16, 16)]
            return 0

        lax.fori_loop(0, PSUP, row_body, 0)
        for t in range(NSTRH):
            sl = pl.ds(t * CHH, CHH)
            pltpu.sync_copy(rows_a.at[sl], shared_div.at[sidxa.at[t]], add=True)
            pltpu.sync_copy(nrows_a.at[sl], shared_div.at[didxa.at[t]], add=True)
            pltpu.sync_copy(rows_b.at[sl], shared_div.at[sidxb.at[t]], add=True)
            pltpu.sync_copy(nrows_b.at[sl], shared_div.at[didxb.at[t]], add=True)
        return 0

    lax.fori_loop(0, nitp, super_body, 0)
    plsc.subcore_barrier()

    def dchunk(j, _):
        ck = s + j * NS

        @pl.when(ck < nzch)
        def _():
            pltpu.sync_copy(shared_div.at[pl.ds(ck * zch, zch)], bounce)
            pltpu.sync_copy(bounce, div_out.at[c].at[pl.ds(ck * zch, zch)])

        return 0

    lax.fori_loop(0, njz, dchunk, 0)


def kernel(xn, xe, edge_index, KNopen, KEopen, KNclose, KN, KE):
    nin, n = xn.shape
    e = xe.shape[1]
    cdim = KNopen.shape[0]
    nlayer = KN.shape[0]
    eh = e // 2
    hpw = eh // NW
    nitp = hpw // PSUP
    zch = 80
    ge = eh // EBP
    f32 = jnp.float32

    src1 = edge_index[0]
    dst1 = edge_index[1]
    src2 = src1.reshape(e // CHH, CHH)
    dst2 = dst1.reshape(e // CHH, CHH)

    zc = jnp.zeros_like(KEopen)
    w2a = jnp.concatenate([KEopen, zc], axis=0)
    w2b = jnp.concatenate([zc, KEopen], axis=0)
    zk = jnp.zeros_like(KNclose)
    kca = jnp.concatenate([KNclose, zk], axis=1)
    kcb = jnp.concatenate([zk, KNclose], axis=1)

    xn_cur, table = pl.pallas_call(
        _prologue_body,
        out_shape=(jax.ShapeDtypeStruct((cdim, n), f32),
                   jax.ShapeDtypeStruct((n, cdim), f32)),
    )(xn, KNopen, KN[0])

    mesh = _mesh()
    gather_call = functools.partial(
        pl.kernel,
        functools.partial(_sc_gather_body, eh=eh, hpw=hpw, nitp=nitp),
        out_type=(jax.ShapeDtypeStruct((eh, 2 * cdim), f32),
                  jax.ShapeDtypeStruct((NW, 2, cdim), f32)),
        mesh=mesh,
        scratch_types=[
            pltpu.VMEM((PSUP,), jnp.int32),
            pltpu.VMEM((PSUP,), jnp.int32),
            pltpu.VMEM((PSUP,), jnp.int32),
            pltpu.VMEM((PSUP,), jnp.int32),
            pltpu.VMEM((PSUP, cdim), f32),
            pltpu.VMEM((PSUP, cdim), f32),
            pltpu.VMEM((PSUP, cdim), f32),
            pltpu.VMEM((PSUP, cdim), f32),
            pltpu.VMEM((2, cdim), f32),
            pltpu.SemaphoreType.DMA,
        ],
        compiler_params=pltpu.CompilerParams(use_tc_tiling_on_sc=False),
    )
    scatter_call = functools.partial(
        pl.kernel,
        functools.partial(_sc_scatter_body, eh=eh, hpw=hpw, nitp=nitp, n=n, zch=zch),
        out_type=jax.ShapeDtypeStruct((NC, n, cdim), f32),
        mesh=mesh,
        scratch_types=[
            pltpu.VMEM((NSTRH, CHH), jnp.int32),
            pltpu.VMEM((NSTRH, CHH), jnp.int32),
            pltpu.VMEM((NSTRH, CHH), jnp.int32),
            pltpu.VMEM((NSTRH, CHH), jnp.int32),
            pltpu.VMEM((PSUP, cdim), f32),
            pltpu.VMEM((PSUP, cdim), f32),
            pltpu.VMEM((PSUP, cdim), f32),
            pltpu.VMEM((PSUP, cdim), f32),
            pltpu.VMEM((zch, cdim), f32),
            pltpu.SemaphoreType.DMA,
            pltpu.VMEM_SHARED((n, cdim), f32),
        ],
        compiler_params=pltpu.CompilerParams(use_tc_tiling_on_sc=False),
    )

    xe_p = None
    xe_out = None
    for i in range(nlayer):
        last = i == nlayer - 1

        ai_p, stats = gather_call()(src1, dst1, table)

        stats_spec = pl.BlockSpec((NW, 2, cdim), lambda i_: (0, 0, 0))
        ebp_spec = pl.BlockSpec((EBP, 2 * cdim), lambda i_: (i_, 0))
        wc_spec = pl.BlockSpec((cdim, 2 * cdim), lambda i_: (0, 0))
        if i == 0:
            xe_p = pl.pallas_call(
                functools.partial(_update0_body, e_total=float(e)),
                grid=(ge,),
                in_specs=[
                    ebp_spec,
                    pl.BlockSpec((nin, EBP), lambda i_: (0, i_)),
                    pl.BlockSpec((nin, EBP), lambda i_: (0, i_ + ge)),
                    pl.BlockSpec((2 * cdim, nin), lambda i_: (0, 0)),
                    pl.BlockSpec((2 * cdim, nin), lambda i_: (0, 0)),
                    stats_spec,
                ],
                out_specs=ebp_spec,
                out_shape=jax.ShapeDtypeStruct((eh, 2 * cdim), f32),
            )(ai_p, xe, xe, w2a, w2b, stats)
        else:
            xe_p, ca, cb = pl.pallas_call(
                functools.partial(_update_close_body, e_total=float(e)),
                grid=(ge,),
                in_specs=[ebp_spec, ebp_spec, stats_spec, wc_spec, wc_spec],
                out_specs=[ebp_spec,
                           pl.BlockSpec((cdim, EBP), lambda i_: (0, i_)),
                           pl.BlockSpec((cdim, EBP), lambda i_: (0, i_))],
                out_shape=(jax.ShapeDtypeStruct((eh, 2 * cdim), f32),
                           jax.ShapeDtypeStruct((cdim, eh), f32),
                           jax.ShapeDtypeStruct((cdim, eh), f32)),
            )(ai_p, xe_p, stats, kca, kcb)
            xe_out = jnp.concatenate([ca, cb], axis=1)

        div_parts = scatter_call()(xe_p, src2, dst2)

        wnext = KNclose if last else KN[i + 1]
        nxt_shape = (cdim, n) if last else (n, cdim)
        xn_cur, nxt = pl.pallas_call(
            functools.partial(_node_body, last=last),
            out_shape=(jax.ShapeDtypeStruct((cdim, n), f32),
                       jax.ShapeDtypeStruct(nxt_shape, f32)),
        )(div_parts, xn_cur, KE[i], wnext)
        if last:
            xn_out = nxt
        else:
            table = nxt

    return (xn_out, xe_out)


# speedup vs baseline: 6.9940x; 1.3471x over previous
import functools

import jax
import jax.numpy as jnp
from jax import lax
from jax.experimental import pallas as pl
from jax.experimental.pallas import tpu as pltpu
from jax.experimental.pallas import tpu_sc as plsc

H = 0.1
EPS = 1e-3

NC = 2
NS = 16
NW = NC * NS

PSUP = 200
CHH = 40
NSTRH = PSUP // CHH

EBP = 1280


def _mesh():
    return plsc.VectorSubcoreMesh(
        core_axis_name="c", subcore_axis_name="s", num_cores=NC, num_subcores=NS
    )


def _prologue_body(xn_ref, knopen_ref, kn0_ref, xn0_ref, y1t_ref):
    xn0 = lax.dot_general(knopen_ref[...], xn_ref[...], (((1,), (0,)), ((), ())),
                          preferred_element_type=jnp.float32)
    xn0_ref[...] = xn0
    y1t_ref[...] = lax.dot_general(xn0, kn0_ref[...], (((0,), (1,)), ((), ())),
                                   preferred_element_type=jnp.float32)


def _stats_mi2(stats, e_total):
    s = jnp.sum(stats, axis=0)
    m = s[0] / e_total
    inv = lax.rsqrt(s[1] - e_total * m * m + EPS)
    return jnp.concatenate([m, m]), jnp.concatenate([inv, inv])


def _update0_body(ai_ref, xea_ref, xeb_ref, w2a_ref, w2b_ref, stats_ref, out_ref,
                  *, e_total):
    m2, inv2 = _stats_mi2(stats_ref[...], e_total)
    xe0 = lax.dot_general(xea_ref[...], w2a_ref[...], (((0,), (1,)), ((), ())),
                          preferred_element_type=jnp.float32)
    xe0 = xe0 + lax.dot_general(xeb_ref[...], w2b_ref[...], (((0,), (1,)), ((), ())),
                                preferred_element_type=jnp.float32)
    a = (ai_ref[...] - m2[None, :]) * inv2[None, :]
    out_ref[...] = xe0 + H * jnp.maximum(a, 0.0)


def _update_close_body(ai_ref, xe_ref, stats_ref, kca_ref, kcb_ref,
                       out_ref, outa_ref, outb_ref, *, e_total):
    m2, inv2 = _stats_mi2(stats_ref[...], e_total)
    a = (ai_ref[...] - m2[None, :]) * inv2[None, :]
    xe2 = xe_ref[...] + H * jnp.maximum(a, 0.0)
    out_ref[...] = xe2
    outa_ref[...] = lax.dot_general(kca_ref[...], xe2, (((1,), (1,)), ((), ())),
                                    preferred_element_type=jnp.float32)
    outb_ref[...] = lax.dot_general(kcb_ref[...], xe2, (((1,), (1,)), ((), ())),
                                    preferred_element_type=jnp.float32)


def _node_body(div_ref, xn_ref, ke_ref, wnext_ref, xn_new_ref, nxt_ref, *, last):
    dsum = div_ref[0] + div_ref[1]
    bi = lax.dot_general(ke_ref[...], dsum, (((1,), (1,)), ((), ())),
                         preferred_element_type=jnp.float32)
    bi = jnp.maximum(bi, 0.0)
    mu = jnp.mean(bi, axis=1, keepdims=True)
    xc = bi - mu
    bn = xc * lax.rsqrt(jnp.sum(xc * xc, axis=1, keepdims=True) + EPS)
    xn_new = xn_ref[...] + H * jnp.maximum(bn, 0.0)
    xn_new_ref[...] = xn_new
    if last:
        nxt_ref[...] = lax.dot_general(wnext_ref[...], xn_new, (((1,), (0,)), ((), ())),
                                       preferred_element_type=jnp.float32)
    else:
        nxt_ref[...] = lax.dot_general(xn_new, wnext_ref[...], (((0,), (1,)), ((), ())),
                                       preferred_element_type=jnp.float32)


def _sc_gather_body(src1, dst1, table, ai_out, stats_out,
                    sidxa, didxa, sidxb, didxb,
                    srows_a, drows_a, srows_b, drows_b, statbuf, sem0, sem1,
                    *, eh, hpw, nitp):
    c = lax.axis_index("c")
    s = lax.axis_index("s")
    wid = s * NC + c
    zero = jnp.zeros((16,), jnp.float32)
    sems = (sem0, sem1)
    rowbufs = (srows_a, drows_a, srows_b, drows_b)

    def fire(k, b):
        pr = wid * hpw + k * PSUP
        pltpu.sync_copy(src1.at[pl.ds(pr, PSUP)], sidxa.at[b])
        pltpu.sync_copy(dst1.at[pl.ds(pr, PSUP)], didxa.at[b])
        pltpu.sync_copy(src1.at[pl.ds(eh + pr, PSUP)], sidxb.at[b])
        pltpu.sync_copy(dst1.at[pl.ds(eh + pr, PSUP)], didxb.at[b])
        for t in range(NSTRH):
            sl = pl.ds(t * CHH, CHH)
            osl = pl.ds(b * PSUP + t * CHH, CHH)
            pltpu.async_copy(table.at[sidxa.at[b].at[sl]], srows_a.at[osl], sems[b])
            pltpu.async_copy(table.at[didxa.at[b].at[sl]], drows_a.at[osl], sems[b])
            pltpu.async_copy(table.at[sidxb.at[b].at[sl]], srows_b.at[osl], sems[b])
            pltpu.async_copy(table.at[didxb.at[b].at[sl]], drows_b.at[osl], sems[b])

    def drain(b):
        dummy = ai_out.at[pl.ds(0, PSUP), pl.ds(0, 64)]
        for buf in rowbufs:
            pltpu.make_async_copy(dummy, buf.at[pl.ds(b * PSUP, PSUP)],
                                  sems[b]).wait()

    def compute(k, b, carry):
        off = b * PSUP

        def row_body(r, cr):
            out = list(cr)
            for sb, db in ((srows_a, drows_a), (srows_b, drows_b)):
                for q in range(4):
                    d = sb[off + r, pl.ds(q * 16, 16)] - db[off + r, pl.ds(q * 16, 16)]
                    sb[off + r, pl.ds(q * 16, 16)] = d
                    out[q] = out[q] + d
                    out[4 + q] = out[4 + q] + d * d
            return tuple(out)

        carry = lax.fori_loop(0, PSUP, row_body, carry)
        pr = wid * hpw + k * PSUP
        pltpu.sync_copy(srows_a.at[pl.ds(off, PSUP)],
                        ai_out.at[pl.ds(pr, PSUP), pl.ds(0, 64)])
        pltpu.sync_copy(srows_b.at[pl.ds(off, PSUP)],
                        ai_out.at[pl.ds(pr, PSUP), pl.ds(64, 64)])
        return carry

    fire(0, 0)

    def body2(it, carry):
        j = it * 2
        fire(j + 1, 1)
        drain(0)
        carry = compute(j, 0, carry)
        fire(j + 2, 0)
        drain(1)
        carry = compute(j + 1, 1, carry)
        return carry

    carry = lax.fori_loop(0, (nitp - 1) // 2, body2, (zero,) * 8)
    drain(0)
    carry = compute(nitp - 1, 0, carry)
    for q in range(4):
        statbuf[0, pl.ds(q * 16, 16)] = carry[q]
        statbuf[1, pl.ds(q * 16, 16)] = carry[4 + q]
    pltpu.sync_copy(statbuf, stats_out.at[wid])


def _sc_scatter_body(xe_p, src2, dst2, div_out,
                     sidxa, didxa, sidxb, didxb,
                     rows_a, rows_b, nrows_a, nrows_b, bounce, sem, shared_div,
                     *, eh, hpw, nitp, n, zch):
    c = lax.axis_index("c")
    s = lax.axis_index("s")
    wid = s * NC + c
    zero = jnp.zeros((16,), jnp.float32)
    nzch = n // zch
    njz = (nzch + NS - 1) // NS

    def zrow(r, _):
        for q in range(4):
            bounce[r, pl.ds(q * 16, 16)] = zero
        return 0

    lax.fori_loop(0, zch, zrow, 0)

    def zchunk(j, _):
        ck = s + j * NS

        @pl.when(ck < nzch)
        def _():
            pltpu.sync_copy(bounce, shared_div.at[pl.ds(ck * zch, zch)])

        return 0

    lax.fori_loop(0, njz, zchunk, 0)
    plsc.subcore_barrier()

    def super_body(i, _):
        pr = wid * hpw + i * PSUP
        rra = pr // CHH
        rrb = (eh + pr) // CHH
        lds = [
            pltpu.async_copy(src2.at[pl.ds(rra, NSTRH)], sidxa, sem),
            pltpu.async_copy(dst2.at[pl.ds(rra, NSTRH)], didxa, sem),
            pltpu.async_copy(src2.at[pl.ds(rrb, NSTRH)], sidxb, sem),
            pltpu.async_copy(dst2.at[pl.ds(rrb, NSTRH)], didxb, sem),
            pltpu.async_copy(xe_p.at[pl.ds(pr, PSUP), pl.ds(0, 64)], rows_a, sem),
            pltpu.async_copy(xe_p.at[pl.ds(pr, PSUP), pl.ds(64, 64)], rows_b, sem),
        ]
        for cp in lds:
            cp.wait()

        def row_body(r, _):
            for rb, nb in ((rows_a, nrows_a), (rows_b, nrows_b)):
                for q in range(4):
                    nb[r, pl.ds(q * 16, 16)] = -rb[r, pl.ds(q * 16, 16)]
            return 0

        lax.fori_loop(0, PSUP, row_body, 0)
        scs = []
        for t in range(NSTRH):
            sl = pl.ds(t * CHH, CHH)
            scs.append(pltpu.async_copy(rows_a.at[sl], shared_div.at[sidxa.at[t]],
                                        sem, add=True))
            scs.append(pltpu.async_copy(nrows_a.at[sl], shared_div.at[didxa.at[t]],
                                        sem, add=True))
            scs.append(pltpu.async_copy(rows_b.at[sl], shared_div.at[sidxb.at[t]],
                                        sem, add=True))
            scs.append(pltpu.async_copy(nrows_b.at[sl], shared_div.at[didxb.at[t]],
                                        sem, add=True))
        for cp in scs:
            cp.wait()
        return 0

    lax.fori_loop(0, nitp, super_body, 0)
    plsc.subcore_barrier()

    def dchunk(j, _):
        ck = s + j * NS

        @pl.when(ck < nzch)
        def _():
            pltpu.sync_copy(shared_div.at[pl.ds(ck * zch, zch)], bounce)
            pltpu.sync_copy(bounce, div_out.at[c].at[pl.ds(ck * zch, zch)])

        return 0

    lax.fori_loop(0, njz, dchunk, 0)


def kernel(xn, xe, edge_index, KNopen, KEopen, KNclose, KN, KE):
    nin, n = xn.shape
    e = xe.shape[1]
    cdim = KNopen.shape[0]
    nlayer = KN.shape[0]
    eh = e // 2
    hpw = eh // NW
    nitp = hpw // PSUP
    zch = 80
    ge = eh // EBP
    f32 = jnp.float32

    src1 = edge_index[0]
    dst1 = edge_index[1]
    src2 = src1.reshape(e // CHH, CHH)
    dst2 = dst1.reshape(e // CHH, CHH)

    zc = jnp.zeros_like(KEopen)
    w2a = jnp.concatenate([KEopen, zc], axis=0)
    w2b = jnp.concatenate([zc, KEopen], axis=0)
    zk = jnp.zeros_like(KNclose)
    kca = jnp.concatenate([KNclose, zk], axis=1)
    kcb = jnp.concatenate([zk, KNclose], axis=1)

    xn_cur, table = pl.pallas_call(
        _prologue_body,
        out_shape=(jax.ShapeDtypeStruct((cdim, n), f32),
                   jax.ShapeDtypeStruct((n, cdim), f32)),
    )(xn, KNopen, KN[0])

    mesh = _mesh()
    gather_call = functools.partial(
        pl.kernel,
        functools.partial(_sc_gather_body, eh=eh, hpw=hpw, nitp=nitp),
        out_type=(jax.ShapeDtypeStruct((eh, 2 * cdim), f32),
                  jax.ShapeDtypeStruct((NW, 2, cdim), f32)),
        mesh=mesh,
        scratch_types=[
            pltpu.VMEM((2, PSUP), jnp.int32),
            pltpu.VMEM((2, PSUP), jnp.int32),
            pltpu.VMEM((2, PSUP), jnp.int32),
            pltpu.VMEM((2, PSUP), jnp.int32),
            pltpu.VMEM((2 * PSUP, cdim), f32),
            pltpu.VMEM((2 * PSUP, cdim), f32),
            pltpu.VMEM((2 * PSUP, cdim), f32),
            pltpu.VMEM((2 * PSUP, cdim), f32),
            pltpu.VMEM((2, cdim), f32),
            pltpu.SemaphoreType.DMA,
            pltpu.SemaphoreType.DMA,
        ],
        compiler_params=pltpu.CompilerParams(use_tc_tiling_on_sc=False),
    )
    scatter_call = functools.partial(
        pl.kernel,
        functools.partial(_sc_scatter_body, eh=eh, hpw=hpw, nitp=nitp, n=n, zch=zch),
        out_type=jax.ShapeDtypeStruct((NC, n, cdim), f32),
        mesh=mesh,
        scratch_types=[
            pltpu.VMEM((NSTRH, CHH), jnp.int32),
            pltpu.VMEM((NSTRH, CHH), jnp.int32),
            pltpu.VMEM((NSTRH, CHH), jnp.int32),
            pltpu.VMEM((NSTRH, CHH), jnp.int32),
            pltpu.VMEM((PSUP, cdim), f32),
            pltpu.VMEM((PSUP, cdim), f32),
            pltpu.VMEM((PSUP, cdim), f32),
            pltpu.VMEM((PSUP, cdim), f32),
            pltpu.VMEM((zch, cdim), f32),
            pltpu.SemaphoreType.DMA,
            pltpu.VMEM_SHARED((n, cdim), f32),
        ],
        compiler_params=pltpu.CompilerParams(use_tc_tiling_on_sc=False),
    )

    xe_p = None
    xe_out = None
    for i in range(nlayer):
        last = i == nlayer - 1

        ai_p, stats = gather_call()(src1, dst1, table)

        stats_spec = pl.BlockSpec((NW, 2, cdim), lambda i_: (0, 0, 0))
        ebp_spec = pl.BlockSpec((EBP, 2 * cdim), lambda i_: (i_, 0))
        wc_spec = pl.BlockSpec((cdim, 2 * cdim), lambda i_: (0, 0))
        if i == 0:
            xe_p = pl.pallas_call(
                functools.partial(_update0_body, e_total=float(e)),
                grid=(ge,),
                in_specs=[
                    ebp_spec,
                    pl.BlockSpec((nin, EBP), lambda i_: (0, i_)),
                    pl.BlockSpec((nin, EBP), lambda i_: (0, i_ + ge)),
                    pl.BlockSpec((2 * cdim, nin), lambda i_: (0, 0)),
                    pl.BlockSpec((2 * cdim, nin), lambda i_: (0, 0)),
                    stats_spec,
                ],
                out_specs=ebp_spec,
                out_shape=jax.ShapeDtypeStruct((eh, 2 * cdim), f32),
            )(ai_p, xe, xe, w2a, w2b, stats)
        else:
            xe_p, ca, cb = pl.pallas_call(
                functools.partial(_update_close_body, e_total=float(e)),
                grid=(ge,),
                in_specs=[ebp_spec, ebp_spec, stats_spec, wc_spec, wc_spec],
                out_specs=[ebp_spec,
                           pl.BlockSpec((cdim, EBP), lambda i_: (0, i_)),
                           pl.BlockSpec((cdim, EBP), lambda i_: (0, i_))],
                out_shape=(jax.ShapeDtypeStruct((eh, 2 * cdim), f32),
                           jax.ShapeDtypeStruct((cdim, eh), f32),
                           jax.ShapeDtypeStruct((cdim, eh), f32)),
            )(ai_p, xe_p, stats, kca, kcb)
            xe_out = jnp.concatenate([ca, cb], axis=1)

        div_parts = scatter_call()(xe_p, src2, dst2)

        wnext = KNclose if last else KN[i + 1]
        nxt_shape = (cdim, n) if last else (n, cdim)
        xn_cur, nxt = pl.pallas_call(
            functools.partial(_node_body, last=last),
            out_shape=(jax.ShapeDtypeStruct((cdim, n), f32),
                       jax.ShapeDtypeStruct(nxt_shape, f32)),
        )(div_parts, xn_cur, KE[i], wnext)
        if last:
            xn_out = nxt
        else:
            table = nxt

    return (xn_out, xe_out)


# speedup vs baseline: 7.7781x; 1.1121x over previous
import functools

import jax
import jax.numpy as jnp
from jax import lax
from jax.experimental import pallas as pl
from jax.experimental.pallas import tpu as pltpu
from jax.experimental.pallas import tpu_sc as plsc

H = 0.1
EPS = 1e-3

NC = 2
NS = 16
NW = NC * NS

PSUP = 200
CHH = 40
NSTRH = PSUP // CHH

EBP = 3200


def _mesh():
    return plsc.VectorSubcoreMesh(
        core_axis_name="c", subcore_axis_name="s", num_cores=NC, num_subcores=NS
    )


def _prologue_body(xn_ref, knopen_ref, kn0_ref, xn0_ref, y1t_ref):
    xn0 = lax.dot_general(knopen_ref[...], xn_ref[...], (((1,), (0,)), ((), ())),
                          preferred_element_type=jnp.float32)
    xn0_ref[...] = xn0
    y1t_ref[...] = lax.dot_general(xn0, kn0_ref[...], (((0,), (1,)), ((), ())),
                                   preferred_element_type=jnp.float32)


def _stats_mi2(stats, e_total):
    s = jnp.sum(stats, axis=0)
    m = s[0] / e_total
    inv = lax.rsqrt(s[1] - e_total * m * m + EPS)
    return jnp.concatenate([m, m]), jnp.concatenate([inv, inv])


def _update0_body(ai_ref, xea_ref, xeb_ref, w2a_ref, w2b_ref, stats_ref, out_ref,
                  *, e_total):
    m2, inv2 = _stats_mi2(stats_ref[...], e_total)
    xe0 = lax.dot_general(xea_ref[...], w2a_ref[...], (((0,), (1,)), ((), ())),
                          preferred_element_type=jnp.float32)
    xe0 = xe0 + lax.dot_general(xeb_ref[...], w2b_ref[...], (((0,), (1,)), ((), ())),
                                preferred_element_type=jnp.float32)
    a = (ai_ref[...] - m2[None, :]) * inv2[None, :]
    out_ref[...] = xe0 + H * jnp.maximum(a, 0.0)


def _update_close_body(ai_ref, xe_ref, stats_ref, kca_ref, kcb_ref,
                       out_ref, outa_ref, outb_ref, *, e_total):
    m2, inv2 = _stats_mi2(stats_ref[...], e_total)
    a = (ai_ref[...] - m2[None, :]) * inv2[None, :]
    xe2 = xe_ref[...] + H * jnp.maximum(a, 0.0)
    out_ref[...] = xe2
    outa_ref[...] = lax.dot_general(kca_ref[...], xe2, (((1,), (1,)), ((), ())),
                                    preferred_element_type=jnp.float32)
    outb_ref[...] = lax.dot_general(kcb_ref[...], xe2, (((1,), (1,)), ((), ())),
                                    preferred_element_type=jnp.float32)


def _node_body(div_ref, xn_ref, ke_ref, wnext_ref, xn_new_ref, nxt_ref, *, last):
    dsum = div_ref[0] + div_ref[1]
    bi = lax.dot_general(ke_ref[...], dsum, (((1,), (1,)), ((), ())),
                         preferred_element_type=jnp.float32)
    bi = jnp.maximum(bi, 0.0)
    mu = jnp.mean(bi, axis=1, keepdims=True)
    xc = bi - mu
    bn = xc * lax.rsqrt(jnp.sum(xc * xc, axis=1, keepdims=True) + EPS)
    xn_new = xn_ref[...] + H * jnp.maximum(bn, 0.0)
    xn_new_ref[...] = xn_new
    if last:
        nxt_ref[...] = lax.dot_general(wnext_ref[...], xn_new, (((1,), (0,)), ((), ())),
                                       preferred_element_type=jnp.float32)
    else:
        nxt_ref[...] = lax.dot_general(xn_new, wnext_ref[...], (((0,), (1,)), ((), ())),
                                       preferred_element_type=jnp.float32)


def _sc_gather_body(src1, dst1, table, ai_out, stats_out,
                    sidxa, didxa, sidxb, didxb,
                    srows_a, drows_a, srows_b, drows_b, statbuf, sem0, sem1,
                    *, eh, hpw, nitp):
    c = lax.axis_index("c")
    s = lax.axis_index("s")
    wid = s * NC + c
    zero = jnp.zeros((16,), jnp.float32)
    sems = (sem0, sem1)
    rowbufs = (srows_a, drows_a, srows_b, drows_b)

    def fire(k, b):
        pr = wid * hpw + k * PSUP
        pltpu.sync_copy(src1.at[pl.ds(pr, PSUP)], sidxa.at[b])
        pltpu.sync_copy(dst1.at[pl.ds(pr, PSUP)], didxa.at[b])
        pltpu.sync_copy(src1.at[pl.ds(eh + pr, PSUP)], sidxb.at[b])
        pltpu.sync_copy(dst1.at[pl.ds(eh + pr, PSUP)], didxb.at[b])
        for t in range(NSTRH):
            sl = pl.ds(t * CHH, CHH)
            osl = pl.ds(b * PSUP + t * CHH, CHH)
            pltpu.async_copy(table.at[sidxa.at[b].at[sl]], srows_a.at[osl], sems[b])
            pltpu.async_copy(table.at[didxa.at[b].at[sl]], drows_a.at[osl], sems[b])
            pltpu.async_copy(table.at[sidxb.at[b].at[sl]], srows_b.at[osl], sems[b])
            pltpu.async_copy(table.at[didxb.at[b].at[sl]], drows_b.at[osl], sems[b])

    def drain(b):
        dummy = ai_out.at[pl.ds(0, PSUP), pl.ds(0, 64)]
        for buf in rowbufs:
            pltpu.make_async_copy(dummy, buf.at[pl.ds(b * PSUP, PSUP)],
                                  sems[b]).wait()

    def compute(k, b, carry):
        off = b * PSUP

        def row_body(r, cr):
            out = list(cr)
            for sb, db in ((srows_a, drows_a), (srows_b, drows_b)):
                for q in range(4):
                    d = sb[off + r, pl.ds(q * 16, 16)] - db[off + r, pl.ds(q * 16, 16)]
                    sb[off + r, pl.ds(q * 16, 16)] = d
                    out[q] = out[q] + d
                    out[4 + q] = out[4 + q] + d * d
            return tuple(out)

        carry = lax.fori_loop(0, PSUP, row_body, carry)
        pr = wid * hpw + k * PSUP
        pltpu.sync_copy(srows_a.at[pl.ds(off, PSUP)],
                        ai_out.at[pl.ds(pr, PSUP), pl.ds(0, 64)])
        pltpu.sync_copy(srows_b.at[pl.ds(off, PSUP)],
                        ai_out.at[pl.ds(pr, PSUP), pl.ds(64, 64)])
        return carry

    fire(0, 0)

    def body2(it, carry):
        j = it * 2
        fire(j + 1, 1)
        drain(0)
        carry = compute(j, 0, carry)
        fire(j + 2, 0)
        drain(1)
        carry = compute(j + 1, 1, carry)
        return carry

    carry = lax.fori_loop(0, (nitp - 1) // 2, body2, (zero,) * 8)
    drain(0)
    carry = compute(nitp - 1, 0, carry)
    for q in range(4):
        statbuf[0, pl.ds(q * 16, 16)] = carry[q]
        statbuf[1, pl.ds(q * 16, 16)] = carry[4 + q]
    pltpu.sync_copy(statbuf, stats_out.at[wid])


def _sc_scatter_body(xe_p, src2, dst2, div_out,
                     sidxa, didxa, sidxb, didxb,
                     rows_a, rows_b, nrows_a, nrows_b, bounce, sem, shared_div,
                     *, eh, hpw, nitp, n, zch):
    c = lax.axis_index("c")
    s = lax.axis_index("s")
    wid = s * NC + c
    zero = jnp.zeros((16,), jnp.float32)
    nzch = n // zch
    njz = (nzch + NS - 1) // NS

    def zrow(r, _):
        for q in range(4):
            bounce[r, pl.ds(q * 16, 16)] = zero
        return 0

    lax.fori_loop(0, zch, zrow, 0)

    def zchunk(j, _):
        ck = s + j * NS

        @pl.when(ck < nzch)
        def _():
            pltpu.sync_copy(bounce, shared_div.at[pl.ds(ck * zch, zch)])

        return 0

    lax.fori_loop(0, njz, zchunk, 0)
    plsc.subcore_barrier()

    def super_body(i, _):
        pr = wid * hpw + i * PSUP
        rra = pr // CHH
        rrb = (eh + pr) // CHH
        lds = [
            pltpu.async_copy(src2.at[pl.ds(rra, NSTRH)], sidxa, sem),
            pltpu.async_copy(dst2.at[pl.ds(rra, NSTRH)], didxa, sem),
            pltpu.async_copy(src2.at[pl.ds(rrb, NSTRH)], sidxb, sem),
            pltpu.async_copy(dst2.at[pl.ds(rrb, NSTRH)], didxb, sem),
            pltpu.async_copy(xe_p.at[pl.ds(pr, PSUP), pl.ds(0, 64)], rows_a, sem),
            pltpu.async_copy(xe_p.at[pl.ds(pr, PSUP), pl.ds(64, 64)], rows_b, sem),
        ]
        for cp in lds:
            cp.wait()

        def row_body(r, _):
            for rb, nb in ((rows_a, nrows_a), (rows_b, nrows_b)):
                for q in range(4):
                    nb[r, pl.ds(q * 16, 16)] = -rb[r, pl.ds(q * 16, 16)]
            return 0

        lax.fori_loop(0, PSUP, row_body, 0)
        scs = []
        for t in range(NSTRH):
            sl = pl.ds(t * CHH, CHH)
            scs.append(pltpu.async_copy(rows_a.at[sl], shared_div.at[sidxa.at[t]],
                                        sem, add=True))
            scs.append(pltpu.async_copy(nrows_a.at[sl], shared_div.at[didxa.at[t]],
                                        sem, add=True))
            scs.append(pltpu.async_copy(rows_b.at[sl], shared_div.at[sidxb.at[t]],
                                        sem, add=True))
            scs.append(pltpu.async_copy(nrows_b.at[sl], shared_div.at[didxb.at[t]],
                                        sem, add=True))
        for cp in scs:
            cp.wait()
        return 0

    lax.fori_loop(0, nitp, super_body, 0)
    plsc.subcore_barrier()

    def dchunk(j, _):
        ck = s + j * NS

        @pl.when(ck < nzch)
        def _():
            pltpu.sync_copy(shared_div.at[pl.ds(ck * zch, zch)], bounce)
            pltpu.sync_copy(bounce, div_out.at[c].at[pl.ds(ck * zch, zch)])

        return 0

    lax.fori_loop(0, njz, dchunk, 0)


def kernel(xn, xe, edge_index, KNopen, KEopen, KNclose, KN, KE):
    nin, n = xn.shape
    e = xe.shape[1]
    cdim = KNopen.shape[0]
    nlayer = KN.shape[0]
    eh = e // 2
    hpw = eh // NW
    nitp = hpw // PSUP
    zch = 80
    ge = eh // EBP
    f32 = jnp.float32

    src1 = edge_index[0]
    dst1 = edge_index[1]
    src2 = src1.reshape(e // CHH, CHH)
    dst2 = dst1.reshape(e // CHH, CHH)

    zc = jnp.zeros_like(KEopen)
    w2a = jnp.concatenate([KEopen, zc], axis=0)
    w2b = jnp.concatenate([zc, KEopen], axis=0)
    zk = jnp.zeros_like(KNclose)
    kca = jnp.concatenate([KNclose, zk], axis=1)
    kcb = jnp.concatenate([zk, KNclose], axis=1)

    xn_cur, table = pl.pallas_call(
        _prologue_body,
        out_shape=(jax.ShapeDtypeStruct((cdim, n), f32),
                   jax.ShapeDtypeStruct((n, cdim), f32)),
    )(xn, KNopen, KN[0])

    mesh = _mesh()
    gather_call = functools.partial(
        pl.kernel,
        functools.partial(_sc_gather_body, eh=eh, hpw=hpw, nitp=nitp),
        out_type=(jax.ShapeDtypeStruct((eh, 2 * cdim), f32),
                  jax.ShapeDtypeStruct((NW, 2, cdim), f32)),
        mesh=mesh,
        scratch_types=[
            pltpu.VMEM((2, PSUP), jnp.int32),
            pltpu.VMEM((2, PSUP), jnp.int32),
            pltpu.VMEM((2, PSUP), jnp.int32),
            pltpu.VMEM((2, PSUP), jnp.int32),
            pltpu.VMEM((2 * PSUP, cdim), f32),
            pltpu.VMEM((2 * PSUP, cdim), f32),
            pltpu.VMEM((2 * PSUP, cdim), f32),
            pltpu.VMEM((2 * PSUP, cdim), f32),
            pltpu.VMEM((2, cdim), f32),
            pltpu.SemaphoreType.DMA,
            pltpu.SemaphoreType.DMA,
        ],
        compiler_params=pltpu.CompilerParams(use_tc_tiling_on_sc=False),
    )
    scatter_call = functools.partial(
        pl.kernel,
        functools.partial(_sc_scatter_body, eh=eh, hpw=hpw, nitp=nitp, n=n, zch=zch),
        out_type=jax.ShapeDtypeStruct((NC, n, cdim), f32),
        mesh=mesh,
        scratch_types=[
            pltpu.VMEM((NSTRH, CHH), jnp.int32),
            pltpu.VMEM((NSTRH, CHH), jnp.int32),
            pltpu.VMEM((NSTRH, CHH), jnp.int32),
            pltpu.VMEM((NSTRH, CHH), jnp.int32),
            pltpu.VMEM((PSUP, cdim), f32),
            pltpu.VMEM((PSUP, cdim), f32),
            pltpu.VMEM((PSUP, cdim), f32),
            pltpu.VMEM((PSUP, cdim), f32),
            pltpu.VMEM((zch, cdim), f32),
            pltpu.SemaphoreType.DMA,
            pltpu.VMEM_SHARED((n, cdim), f32),
        ],
        compiler_params=pltpu.CompilerParams(use_tc_tiling_on_sc=False),
    )

    xe_p = None
    xe_out = None
    for i in range(nlayer):
        last = i == nlayer - 1

        ai_p, stats = gather_call()(src1, dst1, table)

        stats_spec = pl.BlockSpec((NW, 2, cdim), lambda i_: (0, 0, 0))
        ebp_spec = pl.BlockSpec((EBP, 2 * cdim), lambda i_: (i_, 0))
        wc_spec = pl.BlockSpec((cdim, 2 * cdim), lambda i_: (0, 0))
        if i == 0:
            xe_p = pl.pallas_call(
                functools.partial(_update0_body, e_total=float(e)),
                grid=(ge,),
                in_specs=[
                    ebp_spec,
                    pl.BlockSpec((nin, EBP), lambda i_: (0, i_)),
                    pl.BlockSpec((nin, EBP), lambda i_: (0, i_ + ge)),
                    pl.BlockSpec((2 * cdim, nin), lambda i_: (0, 0)),
                    pl.BlockSpec((2 * cdim, nin), lambda i_: (0, 0)),
                    stats_spec,
                ],
                out_specs=ebp_spec,
                out_shape=jax.ShapeDtypeStruct((eh, 2 * cdim), f32),
            )(ai_p, xe, xe, w2a, w2b, stats)
        else:
            xe_p, ca, cb = pl.pallas_call(
                functools.partial(_update_close_body, e_total=float(e)),
                grid=(ge,),
                in_specs=[ebp_spec, ebp_spec, stats_spec, wc_spec, wc_spec],
                out_specs=[ebp_spec,
                           pl.BlockSpec((cdim, EBP), lambda i_: (0, i_)),
                           pl.BlockSpec((cdim, EBP), lambda i_: (0, i_))],
                out_shape=(jax.ShapeDtypeStruct((eh, 2 * cdim), f32),
                           jax.ShapeDtypeStruct((cdim, eh), f32),
                           jax.ShapeDtypeStruct((cdim, eh), f32)),
            )(ai_p, xe_p, stats, kca, kcb)
            xe_out = jnp.concatenate([ca, cb], axis=1)

        div_parts = scatter_call()(xe_p, src2, dst2)

        wnext = KNclose if last else KN[i + 1]
        nxt_shape = (cdim, n) if last else (n, cdim)
        xn_cur, nxt = pl.pallas_call(
            functools.partial(_node_body, last=last),
            out_shape=(jax.ShapeDtypeStruct((cdim, n), f32),
                       jax.ShapeDtypeStruct(nxt_shape, f32)),
        )(div_parts, xn_cur, KE[i], wnext)
        if last:
            xn_out = nxt
        else:
            table = nxt

    return (xn_out, xe_out)
